# baseline (speedup 1.0000x reference)
import functools

import jax
import jax.numpy as jnp
from jax import lax
from jax.experimental import pallas as pl
from jax.experimental.pallas import tpu as pltpu
from jax.experimental.pallas import tpu_sc as plsc

VOCAB = 1000000
D = 32
B = 4096
L = 200
HP = 2 * L

NC = 2
NS = 16
NW = NC * NS
B_PER_W = B // NW
IDX_PER_W = B_PER_W * HP

CHUNKS = (128, 128, 128, 16)


def _body(idx_hbm, table_hbm, out_hbm, idx_v, rows_a, rows_b, out_v, sem0, sem1):
    wid = lax.axis_index("s") * NC + lax.axis_index("c")
    base = wid * IDX_PER_W
    pltpu.sync_copy(idx_hbm.at[pl.ds(base, IDX_PER_W)], idx_v)

    def copies(b, buf, sem):
        off = b * HP
        cs = []
        pos = 0
        for n in CHUNKS:
            cs.append(pltpu.make_async_copy(
                table_hbm.at[idx_v.at[pl.ds(off + pos, n)]],
                buf.at[pl.ds(pos, n)], sem))
            pos += n
        return cs

    def fire(b, buf, sem):
        for c in copies(b, buf, sem):
            c.start()

    def drain(b, buf, sem):
        for c in copies(b, buf, sem):
            c.wait()

    def pool_row(b, buf):
        def rbody(j, accs):
            a0, a1, a2, a3 = accs
            a0 = a0 + buf[4 * j, :]
            a1 = a1 + buf[4 * j + 1, :]
            a2 = a2 + buf[4 * j + 2, :]
            a3 = a3 + buf[4 * j + 3, :]
            return a0, a1, a2, a3

        z = jnp.zeros((16,), jnp.float32)
        a0, a1, a2, a3 = lax.fori_loop(0, HP // 4, rbody, (z, z, z, z), unroll=4)
        m0 = (a0 + a2) * jnp.float32(1.0 / L)
        m1 = (a1 + a3) * jnp.float32(1.0 / L)
        ss = plsc.cumsum(m0 * m0 + m1 * m1)[15]
        i = lax.bitcast_convert_type(ss, jnp.int32)
        i = jnp.int32(0x5F3759DF) - lax.shift_right_logical(i, 1)
        y = lax.bitcast_convert_type(i, jnp.float32)
        for _ in range(3):
            y = y * (jnp.float32(1.5) - jnp.float32(0.5) * ss * y * y)
        d = jnp.maximum(ss * y, jnp.float32(1e-12))
        out_v[b, pl.ds(0, 16)] = m0 / d
        out_v[b, pl.ds(16, 16)] = m1 / d

    fire(0, rows_a, sem0)
    fire(1, rows_b, sem1)

    def step(g, carry):
        b0 = 2 * g
        b1 = b0 + 1
        drain(b0, rows_a, sem0)
        pool_row(b0, rows_a)

        @pl.when(b0 + 2 < B_PER_W)
        def _():
            fire(b0 + 2, rows_a, sem0)

        drain(b1, rows_b, sem1)
        pool_row(b1, rows_b)

        @pl.when(b1 + 2 < B_PER_W)
        def _():
            fire(b1 + 2, rows_b, sem1)

        return carry

    lax.fori_loop(0, B_PER_W // 2, step, 0)
    pltpu.sync_copy(out_v, out_hbm.at[pl.ds(wid * B_PER_W, B_PER_W)])


_embed_pool = functools.partial(
    pl.kernel,
    out_type=jax.ShapeDtypeStruct((B, D), jnp.float32),
    mesh=plsc.VectorSubcoreMesh(
        core_axis_name="c", subcore_axis_name="s", num_cores=NC, num_subcores=NS),
    compiler_params=pltpu.CompilerParams(
        needs_layout_passes=False, use_tc_tiling_on_sc=False),
    scratch_types=[
        pltpu.VMEM((IDX_PER_W,), jnp.int32),
        pltpu.VMEM((HP, 16), jnp.float32),
        pltpu.VMEM((HP, 16), jnp.float32),
        pltpu.VMEM((B_PER_W, D), jnp.float32),
        pltpu.SemaphoreType.DMA,
        pltpu.SemaphoreType.DMA,
    ],
)(_body)


def kernel(x, table):
    xi = jnp.reshape(x.astype(jnp.int32), (B, L))
    xp = jnp.reshape(jnp.stack([2 * xi, 2 * xi + 1], axis=-1), (B * HP,))
    th = jnp.reshape(table, (2 * VOCAB, 16))
    return _embed_pool(xp, th)


# speedup vs baseline: 2.1846x; 2.1846x over previous
import functools

import jax
import jax.numpy as jnp
from jax import lax
from jax.experimental import pallas as pl
from jax.experimental.pallas import tpu as pltpu
from jax.experimental.pallas import tpu_sc as plsc

D = 32
B = 4096
L = 200

NC = 2
NS = 16
NW = NC * NS
B_PER_W = B // NW
IDX_PER_W = B_PER_W * L

C0 = 96
C1 = 104


def _body(idx_hbm, table_hbm, out_hbm, idx_v, rows_a, rows_b, out_v, sem0, sem1):
    wid = lax.axis_index("s") * NC + lax.axis_index("c")
    base = wid * IDX_PER_W
    pltpu.sync_copy(idx_hbm.at[pl.ds(base, IDX_PER_W)], idx_v)

    def copies(b, buf, sem):
        off = b * L
        c0 = pltpu.make_async_copy(
            table_hbm.at[idx_v.at[pl.ds(off, C0)]], buf.at[pl.ds(0, C0)], sem)
        c1 = pltpu.make_async_copy(
            table_hbm.at[idx_v.at[pl.ds(off + C0, C1)]], buf.at[pl.ds(C0, C1)], sem)
        return c0, c1

    def fire(b, buf, sem):
        c0, c1 = copies(b, buf, sem)
        c0.start()
        c1.start()

    def drain(b, buf, sem):
        c0, c1 = copies(b, buf, sem)
        c0.wait()
        c1.wait()

    def pool_row(b, buf):
        def rbody(j, accs):
            a0, a1, a2, a3 = accs
            a0 = a0 + buf[2 * j, pl.ds(0, 16)]
            a1 = a1 + buf[2 * j, pl.ds(16, 16)]
            a2 = a2 + buf[2 * j + 1, pl.ds(0, 16)]
            a3 = a3 + buf[2 * j + 1, pl.ds(16, 16)]
            return a0, a1, a2, a3

        z = jnp.zeros((16,), jnp.float32)
        a0, a1, a2, a3 = lax.fori_loop(0, L // 2, rbody, (z, z, z, z), unroll=4)
        m0 = (a0 + a2) * jnp.float32(1.0 / L)
        m1 = (a1 + a3) * jnp.float32(1.0 / L)
        ss = plsc.cumsum(m0 * m0 + m1 * m1)[15]
        i = lax.bitcast_convert_type(ss, jnp.int32)
        i = jnp.int32(0x5F3759DF) - lax.shift_right_logical(i, 1)
        y = lax.bitcast_convert_type(i, jnp.float32)
        for _ in range(3):
            y = y * (jnp.float32(1.5) - jnp.float32(0.5) * ss * y * y)
        d = jnp.maximum(ss * y, jnp.float32(1e-12))
        out_v[b, pl.ds(0, 16)] = m0 / d
        out_v[b, pl.ds(16, 16)] = m1 / d

    fire(0, rows_a, sem0)
    fire(1, rows_b, sem1)

    def step(g, carry):
        b0 = 2 * g
        b1 = b0 + 1
        drain(b0, rows_a, sem0)
        pool_row(b0, rows_a)

        @pl.when(b0 + 2 < B_PER_W)
        def _():
            fire(b0 + 2, rows_a, sem0)

        drain(b1, rows_b, sem1)
        pool_row(b1, rows_b)

        @pl.when(b1 + 2 < B_PER_W)
        def _():
            fire(b1 + 2, rows_b, sem1)

        return carry

    lax.fori_loop(0, B_PER_W // 2, step, 0)
    pltpu.sync_copy(out_v, out_hbm.at[pl.ds(wid * B_PER_W, B_PER_W)])


_embed_pool = functools.partial(
    pl.kernel,
    out_type=jax.ShapeDtypeStruct((B, D), jnp.float32),
    mesh=plsc.VectorSubcoreMesh(
        core_axis_name="c", subcore_axis_name="s", num_cores=NC, num_subcores=NS),
    compiler_params=pltpu.CompilerParams(
        needs_layout_passes=False, use_tc_tiling_on_sc=False),
    scratch_types=[
        pltpu.VMEM((IDX_PER_W,), jnp.int32),
        pltpu.VMEM((L, D), jnp.float32),
        pltpu.VMEM((L, D), jnp.float32),
        pltpu.VMEM((B_PER_W, D), jnp.float32),
        pltpu.SemaphoreType.DMA,
        pltpu.SemaphoreType.DMA,
    ],
)(_body)


VOCAB = 1000000
RBQ = 2000


def _copy_body(t_ref, o_ref):
    o_ref[...] = jnp.reshape(t_ref[...], (RBQ * 128,))


_to_linear = pl.pallas_call(
    _copy_body,
    grid=(VOCAB * D // 128 // RBQ,),
    in_specs=[pl.BlockSpec((RBQ, 128), lambda i: (i, 0))],
    out_specs=pl.BlockSpec((RBQ * 128,), lambda i: (i,)),
    out_shape=jax.ShapeDtypeStruct((VOCAB * D,), jnp.float32),
)


def kernel(x, table):
    xf = jnp.reshape(x.astype(jnp.int32), (B * L,))
    tlin = jnp.reshape(_to_linear(jnp.reshape(table, (VOCAB * D // 128, 128))),
                       (VOCAB, D))
    return _embed_pool(xf, tlin)


# speedup vs baseline: 2.6991x; 1.2355x over previous
import functools

import jax
import jax.numpy as jnp
from jax import lax
from jax.experimental import pallas as pl
from jax.experimental.pallas import tpu as pltpu
from jax.experimental.pallas import tpu_sc as plsc

D = 32
B = 4096
L = 200

NC = 2
NS = 16
NW = NC * NS
B_PER_W = B // NW
IDX_PER_W = B_PER_W * L

C0 = 96
C1 = 104


def _body(idx_hbm, table_hbm, out_hbm, idx_v, rows_a, rows_b, out_v, sem0, sem1):
    wid = lax.axis_index("s") * NC + lax.axis_index("c")
    base = wid * IDX_PER_W
    pltpu.sync_copy(idx_hbm.at[pl.ds(base, IDX_PER_W)], idx_v)

    def copies(b, buf, sem):
        off = b * L
        c0 = pltpu.make_async_copy(
            table_hbm.at[idx_v.at[pl.ds(off, C0)]], buf.at[pl.ds(0, C0)], sem)
        c1 = pltpu.make_async_copy(
            table_hbm.at[idx_v.at[pl.ds(off + C0, C1)]], buf.at[pl.ds(C0, C1)], sem)
        return c0, c1

    def fire(b, buf, sem):
        c0, c1 = copies(b, buf, sem)
        c0.start()
        c1.start()

    def drain(b, buf, sem):
        c0, c1 = copies(b, buf, sem)
        c0.wait()
        c1.wait()

    def pool_row(b, buf):
        def rbody(j, accs):
            a0, a1, a2, a3 = accs
            a0 = a0 + buf[2 * j, pl.ds(0, 16)]
            a1 = a1 + buf[2 * j, pl.ds(16, 16)]
            a2 = a2 + buf[2 * j + 1, pl.ds(0, 16)]
            a3 = a3 + buf[2 * j + 1, pl.ds(16, 16)]
            return a0, a1, a2, a3

        z = jnp.zeros((16,), jnp.float32)
        a0, a1, a2, a3 = lax.fori_loop(0, L // 2, rbody, (z, z, z, z), unroll=4)
        m0 = (a0 + a2) * jnp.float32(1.0 / L)
        m1 = (a1 + a3) * jnp.float32(1.0 / L)
        ss = plsc.cumsum(m0 * m0 + m1 * m1)[15]
        i = lax.bitcast_convert_type(ss, jnp.int32)
        i = jnp.int32(0x5F3759DF) - lax.shift_right_logical(i, 1)
        y = lax.bitcast_convert_type(i, jnp.float32)
        for _ in range(3):
            y = y * (jnp.float32(1.5) - jnp.float32(0.5) * ss * y * y)
        d = jnp.maximum(ss * y, jnp.float32(1e-12))
        out_v[b, pl.ds(0, 16)] = m0 / d
        out_v[b, pl.ds(16, 16)] = m1 / d

    fire(0, rows_a, sem0)
    fire(1, rows_b, sem1)

    def step(g, carry):
        b0 = 2 * g
        b1 = b0 + 1
        drain(b0, rows_a, sem0)
        pool_row(b0, rows_a)

        @pl.when(b0 + 2 < B_PER_W)
        def _():
            fire(b0 + 2, rows_a, sem0)

        drain(b1, rows_b, sem1)
        pool_row(b1, rows_b)

        @pl.when(b1 + 2 < B_PER_W)
        def _():
            fire(b1 + 2, rows_b, sem1)

        return carry

    lax.fori_loop(0, B_PER_W // 2, step, 0)
    pltpu.sync_copy(out_v, out_hbm.at[pl.ds(wid * B_PER_W, B_PER_W)])


_embed_pool = functools.partial(
    pl.kernel,
    out_type=jax.ShapeDtypeStruct((B, D), jnp.float32),
    mesh=plsc.VectorSubcoreMesh(
        core_axis_name="c", subcore_axis_name="s", num_cores=NC, num_subcores=NS),
    compiler_params=pltpu.CompilerParams(
        needs_layout_passes=False, use_tc_tiling_on_sc=False),
    scratch_types=[
        pltpu.VMEM((IDX_PER_W,), jnp.int32),
        pltpu.VMEM((L, D), jnp.float32),
        pltpu.VMEM((L, D), jnp.float32),
        pltpu.VMEM((B_PER_W, D), jnp.float32),
        pltpu.SemaphoreType.DMA,
        pltpu.SemaphoreType.DMA,
    ],
)(_body)


VOCAB = 1000000
CW = 1664
QW = CW // 4


def _transpose_body(t_ref, o_ref):
    t2 = jnp.transpose(t_ref[...])
    w = jnp.concatenate([t2[k * QW:(k + 1) * QW] for k in range(4)], axis=1)
    o_ref[...] = jnp.reshape(w, (CW * D,))


_to_linear = pl.pallas_call(
    _transpose_body,
    grid=(VOCAB // CW,),
    in_specs=[pl.BlockSpec((D, CW), lambda i: (0, i))],
    out_specs=pl.BlockSpec((CW * D,), lambda i: (i,)),
    out_shape=jax.ShapeDtypeStruct((VOCAB * D,), jnp.float32),
)


def kernel(x, table):
    xi = jnp.reshape(x.astype(jnp.int32), (B * L,))
    u = xi % CW
    xf = (xi - u) + (u % QW) * 4 + u // QW
    tlin = jnp.reshape(_to_linear(table.T), (VOCAB, D))
    return _embed_pool(xf, tlin)


# speedup vs baseline: 2.7080x; 1.0033x over previous
import functools

import jax
import jax.numpy as jnp
from jax import lax
from jax.experimental import pallas as pl
from jax.experimental.pallas import tpu as pltpu
from jax.experimental.pallas import tpu_sc as plsc

D = 32
B = 4096
L = 200

NC = 2
NS = 16
NW = NC * NS
B_PER_W = B // NW
IDX_PER_W = B_PER_W * L

C0 = 96
C1 = 104


def _body(idx_hbm, table_hbm, out_hbm, idx_v, rows_a, rows_b, out_v, sem0, sem1):
    wid = lax.axis_index("s") * NC + lax.axis_index("c")
    base = wid * IDX_PER_W
    pltpu.sync_copy(idx_hbm.at[pl.ds(base, IDX_PER_W)], idx_v)

    def copies(b, buf, sem):
        off = b * L
        c0 = pltpu.make_async_copy(
            table_hbm.at[idx_v.at[pl.ds(off, C0)]], buf.at[pl.ds(0, C0)], sem)
        c1 = pltpu.make_async_copy(
            table_hbm.at[idx_v.at[pl.ds(off + C0, C1)]], buf.at[pl.ds(C0, C1)], sem)
        return c0, c1

    def fire(b, buf, sem):
        c0, c1 = copies(b, buf, sem)
        c0.start()
        c1.start()

    def drain(b, buf, sem):
        c0, c1 = copies(b, buf, sem)
        c0.wait()
        c1.wait()

    def pool_row(b, buf):
        def rbody(j, accs):
            a0, a1, a2, a3 = accs
            a0 = a0 + buf[2 * j, pl.ds(0, 16)]
            a1 = a1 + buf[2 * j, pl.ds(16, 16)]
            a2 = a2 + buf[2 * j + 1, pl.ds(0, 16)]
            a3 = a3 + buf[2 * j + 1, pl.ds(16, 16)]
            return a0, a1, a2, a3

        z = jnp.zeros((16,), jnp.float32)
        a0, a1, a2, a3 = lax.fori_loop(0, L // 2, rbody, (z, z, z, z), unroll=4)
        m0 = (a0 + a2) * jnp.float32(1.0 / L)
        m1 = (a1 + a3) * jnp.float32(1.0 / L)
        ss = plsc.cumsum(m0 * m0 + m1 * m1)[15]
        i = lax.bitcast_convert_type(ss, jnp.int32)
        i = jnp.int32(0x5F3759DF) - lax.shift_right_logical(i, 1)
        y = lax.bitcast_convert_type(i, jnp.float32)
        for _ in range(3):
            y = y * (jnp.float32(1.5) - jnp.float32(0.5) * ss * y * y)
        d = jnp.maximum(ss * y, jnp.float32(1e-12))
        out_v[b, pl.ds(0, 16)] = m0 / d
        out_v[b, pl.ds(16, 16)] = m1 / d

    fire(0, rows_a, sem0)
    fire(1, rows_b, sem1)

    def step(g, carry):
        b0 = 2 * g
        b1 = b0 + 1
        drain(b0, rows_a, sem0)
        pool_row(b0, rows_a)

        @pl.when(b0 + 2 < B_PER_W)
        def _():
            fire(b0 + 2, rows_a, sem0)

        drain(b1, rows_b, sem1)
        pool_row(b1, rows_b)

        @pl.when(b1 + 2 < B_PER_W)
        def _():
            fire(b1 + 2, rows_b, sem1)

        return carry

    lax.fori_loop(0, B_PER_W // 2, step, 0)
    pltpu.sync_copy(out_v, out_hbm.at[pl.ds(wid * B_PER_W, B_PER_W)])


_embed_pool = functools.partial(
    pl.kernel,
    out_type=jax.ShapeDtypeStruct((B, D), jnp.float32),
    mesh=plsc.VectorSubcoreMesh(
        core_axis_name="c", subcore_axis_name="s", num_cores=NC, num_subcores=NS),
    compiler_params=pltpu.CompilerParams(
        needs_layout_passes=False, use_tc_tiling_on_sc=False),
    scratch_types=[
        pltpu.VMEM((IDX_PER_W,), jnp.int32),
        pltpu.VMEM((L, D), jnp.float32),
        pltpu.VMEM((L, D), jnp.float32),
        pltpu.VMEM((B_PER_W, D), jnp.float32),
        pltpu.SemaphoreType.DMA,
        pltpu.SemaphoreType.DMA,
    ],
)(_body)


VOCAB = 1000000
CW = 1664
NBLK = (VOCAB + CW - 1) // CW
VPAD = NBLK * CW
QW = CW // 4


def _transpose_body(t_ref, o_ref):
    t2 = jnp.transpose(t_ref[...])
    w = jnp.concatenate([t2[k * QW:(k + 1) * QW] for k in range(4)], axis=1)
    o_ref[...] = jnp.reshape(w, (CW * D,))


_to_linear = pl.pallas_call(
    _transpose_body,
    grid=(NBLK,),
    in_specs=[pl.BlockSpec((D, CW), lambda i: (0, i))],
    out_specs=pl.BlockSpec((CW * D,), lambda i: (i,)),
    out_shape=jax.ShapeDtypeStruct((VPAD * D,), jnp.float32),
)


def kernel(x, table):
    xi = jnp.reshape(x.astype(jnp.int32), (B * L,))
    u = xi % CW
    xf = (xi - u) + (u % QW) * 4 + u // QW
    tlin = jnp.reshape(_to_linear(table.T), (VPAD, D))
    return _embed_pool(xf, tlin)


# speedup vs baseline: 4.3218x; 1.5959x over previous
import functools

import jax
import jax.numpy as jnp
from jax import lax
from jax.experimental import pallas as pl
from jax.experimental.pallas import tpu as pltpu
from jax.experimental.pallas import tpu_sc as plsc

D = 32
B = 4096
L = 200

NC = 2
NS = 16
NW = NC * NS
B_PER_W = B // NW
IDX_PER_W = B_PER_W * L

C0 = 96
C1 = 104


def _body(idx_hbm, table_hbm, out_hbm, idx_v, rows_a, rows_b, out_v, sem0, sem1):
    wid = lax.axis_index("s") * NC + lax.axis_index("c")
    base = wid * IDX_PER_W
    pltpu.sync_copy(idx_hbm.at[pl.ds(base, IDX_PER_W)], idx_v)

    def copies(b, buf, sem):
        off = b * L
        c0 = pltpu.make_async_copy(
            table_hbm.at[idx_v.at[pl.ds(off, C0)]], buf.at[pl.ds(0, C0)], sem)
        c1 = pltpu.make_async_copy(
            table_hbm.at[idx_v.at[pl.ds(off + C0, C1)]], buf.at[pl.ds(C0, C1)], sem)
        return c0, c1

    def fire(b, buf, sem):
        c0, c1 = copies(b, buf, sem)
        c0.start()
        c1.start()

    def drain(b, buf, sem):
        c0, c1 = copies(b, buf, sem)
        c0.wait()
        c1.wait()

    def pool_row(b, buf):
        def rbody(j, accs):
            a0, a1, a2, a3 = accs
            a0 = a0 + buf[2 * j, pl.ds(0, 16)]
            a1 = a1 + buf[2 * j, pl.ds(16, 16)]
            a2 = a2 + buf[2 * j + 1, pl.ds(0, 16)]
            a3 = a3 + buf[2 * j + 1, pl.ds(16, 16)]
            return a0, a1, a2, a3

        z = jnp.zeros((16,), jnp.float32)
        a0, a1, a2, a3 = lax.fori_loop(0, L // 2, rbody, (z, z, z, z), unroll=4)
        m0 = (a0 + a2) * jnp.float32(1.0 / L)
        m1 = (a1 + a3) * jnp.float32(1.0 / L)
        ss = plsc.cumsum(m0 * m0 + m1 * m1)[15]
        i = lax.bitcast_convert_type(ss, jnp.int32)
        i = jnp.int32(0x5F3759DF) - lax.shift_right_logical(i, 1)
        y = lax.bitcast_convert_type(i, jnp.float32)
        for _ in range(3):
            y = y * (jnp.float32(1.5) - jnp.float32(0.5) * ss * y * y)
        d = jnp.maximum(ss * y, jnp.float32(1e-12))
        out_v[b, pl.ds(0, 16)] = m0 / d
        out_v[b, pl.ds(16, 16)] = m1 / d

    fire(0, rows_a, sem0)
    fire(1, rows_b, sem1)

    def step(g, carry):
        b0 = 2 * g
        b1 = b0 + 1
        drain(b0, rows_a, sem0)
        pool_row(b0, rows_a)

        @pl.when(b0 + 2 < B_PER_W)
        def _():
            fire(b0 + 2, rows_a, sem0)

        drain(b1, rows_b, sem1)
        pool_row(b1, rows_b)

        @pl.when(b1 + 2 < B_PER_W)
        def _():
            fire(b1 + 2, rows_b, sem1)

        return carry

    lax.fori_loop(0, B_PER_W // 2, step, 0)
    pltpu.sync_copy(out_v, out_hbm.at[pl.ds(wid * B_PER_W, B_PER_W)])


_embed_pool = functools.partial(
    pl.kernel,
    out_type=jax.ShapeDtypeStruct((B, D), jnp.float32),
    mesh=plsc.VectorSubcoreMesh(
        core_axis_name="c", subcore_axis_name="s", num_cores=NC, num_subcores=NS),
    compiler_params=pltpu.CompilerParams(
        needs_layout_passes=False, use_tc_tiling_on_sc=False),
    scratch_types=[
        pltpu.VMEM((IDX_PER_W,), jnp.int32),
        pltpu.VMEM((L, D), jnp.float32),
        pltpu.VMEM((L, D), jnp.float32),
        pltpu.VMEM((B_PER_W, D), jnp.float32),
        pltpu.SemaphoreType.DMA,
        pltpu.SemaphoreType.DMA,
    ],
)(_body)


VOCAB = 1000000
CW = 13312
NBLK = (VOCAB + CW - 1) // CW
VPAD = NBLK * CW
QW = CW // 4


def _transpose_body(t_ref, o_ref):
    t2 = jnp.transpose(t_ref[...])
    w = jnp.concatenate([t2[k * QW:(k + 1) * QW] for k in range(4)], axis=1)
    o_ref[...] = jnp.reshape(w, (CW * D,))


_to_linear = pl.pallas_call(
    _transpose_body,
    grid=(NBLK,),
    in_specs=[pl.BlockSpec((D, CW), lambda i: (0, i))],
    out_specs=pl.BlockSpec((CW * D,), lambda i: (i,)),
    out_shape=jax.ShapeDtypeStruct((VPAD * D,), jnp.float32),
)


def kernel(x, table):
    xi = jnp.reshape(x.astype(jnp.int32), (B * L,))
    u = xi % CW
    xf = (xi - u) + (u % QW) * 4 + u // QW
    tlin = jnp.reshape(_to_linear(table.T), (VPAD, D))
    return _embed_pool(xf, tlin)


# speedup vs baseline: 4.3582x; 1.0084x over previous
import functools

import jax
import jax.numpy as jnp
from jax import lax
from jax.experimental import pallas as pl
from jax.experimental.pallas import tpu as pltpu
from jax.experimental.pallas import tpu_sc as plsc

D = 32
B = 4096
L = 200

NC = 2
NS = 16
NW = NC * NS
B_PER_W = B // NW
IDX_PER_W = B_PER_W * L

C0 = 96
C1 = 104


def _body(idx_hbm, table_hbm, out_hbm, idx_v, rows_a, rows_b, out_v, sem0, sem1):
    wid = lax.axis_index("s") * NC + lax.axis_index("c")
    base = wid * IDX_PER_W
    pltpu.sync_copy(idx_hbm.at[pl.ds(base, IDX_PER_W)], idx_v)

    def copies(b, buf, sem):
        off = b * L
        c0 = pltpu.make_async_copy(
            table_hbm.at[idx_v.at[pl.ds(off, C0)]], buf.at[pl.ds(0, C0)], sem)
        c1 = pltpu.make_async_copy(
            table_hbm.at[idx_v.at[pl.ds(off + C0, C1)]], buf.at[pl.ds(C0, C1)], sem)
        return c0, c1

    def fire(b, buf, sem):
        c0, c1 = copies(b, buf, sem)
        c0.start()
        c1.start()

    def drain(b, buf, sem):
        c0, c1 = copies(b, buf, sem)
        c0.wait()
        c1.wait()

    def pool_row(b, buf):
        def rbody(j, accs):
            a0, a1, a2, a3 = accs
            a0 = a0 + buf[2 * j, pl.ds(0, 16)]
            a1 = a1 + buf[2 * j, pl.ds(16, 16)]
            a2 = a2 + buf[2 * j + 1, pl.ds(0, 16)]
            a3 = a3 + buf[2 * j + 1, pl.ds(16, 16)]
            return a0, a1, a2, a3

        z = jnp.zeros((16,), jnp.float32)
        a0, a1, a2, a3 = lax.fori_loop(0, L // 2, rbody, (z, z, z, z), unroll=4)
        m0 = (a0 + a2) * jnp.float32(1.0 / L)
        m1 = (a1 + a3) * jnp.float32(1.0 / L)
        ss = plsc.cumsum(m0 * m0 + m1 * m1)[15]
        i = lax.bitcast_convert_type(ss, jnp.int32)
        i = jnp.int32(0x5F3759DF) - lax.shift_right_logical(i, 1)
        y = lax.bitcast_convert_type(i, jnp.float32)
        for _ in range(3):
            y = y * (jnp.float32(1.5) - jnp.float32(0.5) * ss * y * y)
        d = jnp.maximum(ss * y, jnp.float32(1e-12))
        out_v[b, pl.ds(0, 16)] = m0 / d
        out_v[b, pl.ds(16, 16)] = m1 / d

    fire(0, rows_a, sem0)
    fire(1, rows_b, sem1)

    def step(g, carry):
        b0 = 2 * g
        b1 = b0 + 1
        drain(b0, rows_a, sem0)
        pool_row(b0, rows_a)

        @pl.when(b0 + 2 < B_PER_W)
        def _():
            fire(b0 + 2, rows_a, sem0)

        drain(b1, rows_b, sem1)
        pool_row(b1, rows_b)

        @pl.when(b1 + 2 < B_PER_W)
        def _():
            fire(b1 + 2, rows_b, sem1)

        return carry

    lax.fori_loop(0, B_PER_W // 2, step, 0)
    pltpu.sync_copy(out_v, out_hbm.at[pl.ds(wid * B_PER_W, B_PER_W)])


_embed_pool = functools.partial(
    pl.kernel,
    out_type=jax.ShapeDtypeStruct((B, D), jnp.float32),
    mesh=plsc.VectorSubcoreMesh(
        core_axis_name="c", subcore_axis_name="s", num_cores=NC, num_subcores=NS),
    compiler_params=pltpu.CompilerParams(
        needs_layout_passes=False, use_tc_tiling_on_sc=False),
    scratch_types=[
        pltpu.VMEM((IDX_PER_W,), jnp.int32),
        pltpu.VMEM((L, D), jnp.float32),
        pltpu.VMEM((L, D), jnp.float32),
        pltpu.VMEM((B_PER_W, D), jnp.float32),
        pltpu.SemaphoreType.DMA,
        pltpu.SemaphoreType.DMA,
    ],
)(_body)


VOCAB = 1000000
CW = 26624
NBLK = (VOCAB + CW - 1) // CW
VPAD = NBLK * CW
QW = CW // 4


def _transpose_body(t_ref, o_ref):
    t2 = jnp.transpose(t_ref[...])
    w = jnp.concatenate([t2[k * QW:(k + 1) * QW] for k in range(4)], axis=1)
    o_ref[...] = jnp.reshape(w, (CW * D,))


_to_linear = pl.pallas_call(
    _transpose_body,
    grid=(NBLK,),
    in_specs=[pl.BlockSpec((D, CW), lambda i: (0, i))],
    out_specs=pl.BlockSpec((CW * D,), lambda i: (i,)),
    out_shape=jax.ShapeDtypeStruct((VPAD * D,), jnp.float32),
)


def kernel(x, table):
    xi = jnp.reshape(x.astype(jnp.int32), (B * L,))
    u = xi % CW
    xf = (xi - u) + (u % QW) * 4 + u // QW
    tlin = jnp.reshape(_to_linear(table.T), (VPAD, D))
    return _embed_pool(xf, tlin)
